# Initial kernel scaffold; baseline (speedup 1.0000x reference)
import jax
import jax.numpy as jnp
from jax.experimental import pallas as pl


def kernel(actors, actor_idcs, actor_ctrs, nodes, node_idcs, node_ctrs, params):
    raise NotImplementedError("write your pallas kernel here")


# speedup vs baseline: 21.0373x; 21.0373x over previous
import functools

import jax
import jax.numpy as jnp
from jax import lax
from jax.experimental import pallas as pl
from jax.experimental.pallas import tpu as pltpu
from jax.experimental.pallas import tpu_sc as plsc

NA = 5000
NC = 5000
D = 128
TH = 0.045
NP = 5120
ECAP = 196608
NCORE = 2
NSUB = 16
NW = NCORE * NSUB
ETILE = ECAP // NW
CHUNK = 128
NCHUNK = ETILE // CHUNK
ROWS = NP // NSUB

_HIGH = jax.lax.Precision.HIGHEST


def _gn(x, w, b):
    m = jnp.mean(x, axis=1, keepdims=True)
    v = jnp.mean((x - m) ** 2, axis=1, keepdims=True)
    return (x - m) * jax.lax.rsqrt(v + 1e-5) * w + b


def _mask_body(actr_ref, cctr_ref, out_ref):
    ax = actr_ref[:, 0:1]
    ay = actr_ref[:, 1:2]
    cx = cctr_ref[0:1, :]
    cy = cctr_ref[1:2, :]
    dx = ax - cx
    dy = ay - cy
    dist = jnp.sqrt(dx * dx + dy * dy)
    out_ref[...] = (dist <= TH).astype(jnp.int8)


def _mask_call(actr2, cctr2):
    return pl.pallas_call(
        _mask_body,
        grid=(NP // 256, NP // 512),
        in_specs=[
            pl.BlockSpec((256, 128), lambda i, j: (i, 0)),
            pl.BlockSpec((8, 512), lambda i, j: (0, j)),
        ],
        out_specs=pl.BlockSpec((256, 512), lambda i, j: (i, j)),
        out_shape=jax.ShapeDtypeStruct((NP, NP), jnp.int8),
    )(actr2, cctr2)


def _pre_body(agts_ref, actr_ref, nodes_ref, nctr_ref, wqt_ref, qgw_ref,
              qgb_ref, w1pt_ref, bt_ref, ct_ref, b1_ref, ua_ref, vc_ref):
    agts = agts_ref[...]
    q = jnp.maximum(
        _gn(jnp.dot(agts, wqt_ref[...], precision=_HIGH),
            qgw_ref[...], qgb_ref[...]), 0.0)
    ua_ref[:, :D] = jnp.dot(actr_ref[...], w1pt_ref[...], precision=_HIGH)
    ua_ref[:, D:] = jnp.dot(q, bt_ref[...], precision=_HIGH)
    vc_ref[:, :D] = b1_ref[...] - jnp.dot(nctr_ref[...], w1pt_ref[...],
                                          precision=_HIGH)
    vc_ref[:, D:] = jnp.dot(nodes_ref[...], ct_ref[...], precision=_HIGH)


def _pre_call(agts, actr2, nodes_p, nctr2, wqt, qgw, qgb, w1pt, bt, ct, b1):
    full = lambda i: (0, 0)
    row = lambda i: (i, 0)
    return pl.pallas_call(
        _pre_body,
        grid=(NP // 512,),
        in_specs=[
            pl.BlockSpec((512, D), row),
            pl.BlockSpec((512, D), row),
            pl.BlockSpec((512, D), row),
            pl.BlockSpec((512, D), row),
            pl.BlockSpec((D, D), full),
            pl.BlockSpec((1, D), full),
            pl.BlockSpec((1, D), full),
            pl.BlockSpec((D, D), full),
            pl.BlockSpec((D, D), full),
            pl.BlockSpec((D, D), full),
            pl.BlockSpec((1, D), full),
        ],
        out_specs=[
            pl.BlockSpec((512, 2 * D), row),
            pl.BlockSpec((512, 2 * D), row),
        ],
        out_shape=[
            jax.ShapeDtypeStruct((NP, 2 * D), jnp.float32),
            jax.ShapeDtypeStruct((NP, 2 * D), jnp.float32),
        ],
    )(agts, actr2, nodes_p, nctr2, wqt, qgw, qgb, w1pt, bt, ct, b1)


def _sc_gather(ua, vc, ei2, ej2):
    mesh = plsc.VectorSubcoreMesh(core_axis_name="c", subcore_axis_name="s")

    @functools.partial(
        pl.kernel,
        mesh=mesh,
        out_type=[
            jax.ShapeDtypeStruct((ECAP, 2 * D), jnp.float32),
            jax.ShapeDtypeStruct((ECAP, 2 * D), jnp.float32),
        ],
        scratch_types=[
            pltpu.VMEM((NCHUNK, CHUNK), jnp.int32),
            pltpu.VMEM((NCHUNK, CHUNK), jnp.int32),
            pltpu.VMEM((CHUNK, 2 * D), jnp.float32),
            pltpu.VMEM((CHUNK, 2 * D), jnp.float32),
            pltpu.SemaphoreType.DMA,
            pltpu.SemaphoreType.DMA,
        ],
    )
    def k(ua_hbm, vc_hbm, ei_hbm, ej_hbm, ga_hbm, gc_hbm, ia, ic, ba, bc,
          sa, sb):
        wid = lax.axis_index("s") * NCORE + lax.axis_index("c")
        rbase = wid * NCHUNK
        ebase = wid * ETILE
        pltpu.sync_copy(ei_hbm.at[pl.ds(rbase, NCHUNK)], ia)
        pltpu.sync_copy(ej_hbm.at[pl.ds(rbase, NCHUNK)], ic)

        def body(j, carry):
            ca = pltpu.async_copy(ua_hbm.at[ia.at[j]], ba, sa)
            cb = pltpu.async_copy(vc_hbm.at[ic.at[j]], bc, sb)
            ca.wait()
            cb.wait()
            pltpu.sync_copy(ba, ga_hbm.at[pl.ds(ebase + j * CHUNK, CHUNK)])
            pltpu.sync_copy(bc, gc_hbm.at[pl.ds(ebase + j * CHUNK, CHUNK)])
            return carry

        lax.fori_loop(0, NCHUNK, body, 0)

    return k(ua, vc, ei2, ej2)


def _edge_body(ga_ref, gc_ref, w2t_ref, at_ref, c2t_ref, dgw_ref, dgb_ref,
               cgw_ref, cgb_ref, out_ref):
    ga = ga_ref[...]
    gc = gc_ref[...]
    e1 = jnp.maximum(ga[:, :D] + gc[:, :D], 0.0)
    z = jnp.dot(e1, w2t_ref[...], precision=_HIGH)
    e2 = jnp.maximum(_gn(z, dgw_ref[...], dgb_ref[...]), 0.0)
    h = jnp.dot(e2, at_ref[...], precision=_HIGH) + ga[:, D:] + gc[:, D:]
    cc = jnp.maximum(_gn(h, cgw_ref[...], cgb_ref[...]), 0.0)
    out_ref[...] = jnp.dot(cc, c2t_ref[...], precision=_HIGH)


def _edge_call(ga, gc, w2t, at, c2t, dgw, dgb, cgw, cgb):
    full = lambda i: (0, 0)
    row = lambda i: (i, 0)
    return pl.pallas_call(
        _edge_body,
        grid=(ECAP // 512,),
        in_specs=[
            pl.BlockSpec((512, 2 * D), row),
            pl.BlockSpec((512, 2 * D), row),
            pl.BlockSpec((D, D), full),
            pl.BlockSpec((D, D), full),
            pl.BlockSpec((D, D), full),
            pl.BlockSpec((1, D), full),
            pl.BlockSpec((1, D), full),
            pl.BlockSpec((1, D), full),
            pl.BlockSpec((1, D), full),
        ],
        out_specs=pl.BlockSpec((512, D), row),
        out_shape=jax.ShapeDtypeStruct((ECAP, D), jnp.float32),
    )(ga, gc, w2t, at, c2t, dgw, dgb, cgw, cgb)


def _sc_scatter(oute, sidx2, zrows):
    mesh = plsc.VectorSubcoreMesh(core_axis_name="c", subcore_axis_name="s")

    @functools.partial(
        pl.kernel,
        mesh=mesh,
        out_type=jax.ShapeDtypeStruct((NCORE, NP, D), jnp.float32),
        scratch_types=[
            pltpu.VMEM((NCHUNK, CHUNK), jnp.int32),
            pltpu.VMEM((CHUNK, D), jnp.float32),
            pltpu.VMEM_SHARED((NP, D), jnp.float32),
        ],
    )
    def k(oute_hbm, sidx_hbm, zrows_hbm, acc_hbm, ix, buf, shared):
        cid = lax.axis_index("c")
        sid = lax.axis_index("s")
        wid = sid * NCORE + cid
        pltpu.sync_copy(zrows_hbm, shared.at[pl.ds(sid * ROWS, ROWS)])
        pltpu.sync_copy(sidx_hbm.at[pl.ds(wid * NCHUNK, NCHUNK)], ix)
        plsc.subcore_barrier()

        def body(j, carry):
            pltpu.sync_copy(oute_hbm.at[pl.ds(wid * ETILE + j * CHUNK, CHUNK)],
                            buf)
            pltpu.sync_copy(buf, shared.at[ix.at[j]], add=True)
            return carry

        lax.fori_loop(0, NCHUNK, body, 0)
        plsc.subcore_barrier()
        pltpu.sync_copy(shared.at[pl.ds(sid * ROWS, ROWS)],
                        acc_hbm.at[cid, pl.ds(sid * ROWS, ROWS)])

    return k(oute, sidx2, zrows)


def _post_body(agts_ref, a0_ref, a1_ref, awt_ref, nw_ref, nb_ref, lwt_ref,
               lgw_ref, lgb_ref, out_ref):
    agts = agts_ref[...]
    a = jnp.dot(agts, awt_ref[...], precision=_HIGH) + a0_ref[...] + a1_ref[...]
    a = jnp.maximum(_gn(a, nw_ref[...], nb_ref[...]), 0.0)
    a = _gn(jnp.dot(a, lwt_ref[...], precision=_HIGH), lgw_ref[...],
            lgb_ref[...])
    out_ref[...] = jnp.maximum(a + agts, 0.0)


def _post_call(agts, a0, a1, awt, nw, nb, lwt, lgw, lgb):
    full = lambda i: (0, 0)
    row = lambda i: (i, 0)
    return pl.pallas_call(
        _post_body,
        grid=(NP // 512,),
        in_specs=[
            pl.BlockSpec((512, D), row),
            pl.BlockSpec((512, D), row),
            pl.BlockSpec((512, D), row),
            pl.BlockSpec((D, D), full),
            pl.BlockSpec((1, D), full),
            pl.BlockSpec((1, D), full),
            pl.BlockSpec((D, D), full),
            pl.BlockSpec((1, D), full),
            pl.BlockSpec((1, D), full),
        ],
        out_specs=pl.BlockSpec((512, D), row),
        out_shape=jax.ShapeDtypeStruct((NP, D), jnp.float32),
    )(agts, a0, a1, awt, nw, nb, lwt, lgw, lgb)


def kernel(actors, actor_idcs, actor_ctrs, nodes, node_idcs, node_ctrs,
           params):
    f32 = jnp.float32
    agt_ctrs = actor_ctrs.reshape(-1, 2)
    ctx_ctrs = node_ctrs.reshape(-1, 2)

    actr2 = jnp.full((NP, D), 9.0, f32).at[:NA, :2].set(agt_ctrs)
    actr2 = actr2.at[:NA, 2:].set(0.0)
    cctr2 = jnp.full((8, NP), 9.0, f32).at[:2, :NC].set(ctx_ctrs.T)
    cctr2 = cctr2.at[2:, :].set(0.0)

    mask = _mask_call(actr2, cctr2)
    flat = jnp.nonzero(mask.reshape(-1), size=ECAP,
                       fill_value=NP * NP)[0].astype(jnp.int32)
    valid = flat < NP * NP
    ei = flat // NP
    ej = flat - ei * NP
    ei2 = jnp.where(valid, ei, 0).reshape(ECAP // CHUNK, CHUNK)
    ej2 = jnp.where(valid, ej, 0).reshape(ECAP // CHUNK, CHUNK)
    sidx2 = jnp.where(valid, ei, NP - 1).reshape(ECAP // CHUNK, CHUNK)

    agts = jnp.zeros((NP, D), f32).at[:NA].set(actors)
    nodes_p = jnp.zeros((NP, D), f32).at[:NC].set(nodes)
    nctr2 = jnp.zeros((NP, D), f32).at[:NC, :2].set(ctx_ctrs)
    zrows = jnp.zeros((ROWS, D), f32)

    for i in range(2):
        p = {k: v[i] for k, v in params.items()}
        w1p = jnp.zeros((D, D), f32).at[:, :2].set(p['dist_w1'])
        ua, vc = _pre_call(
            agts, actr2, nodes_p, nctr2,
            p['query_w'].T, p['query_gnw'][None], p['query_gnb'][None],
            w1p.T, p['ctx_w1'][:, D:2 * D].T, p['ctx_w1'][:, 2 * D:].T,
            p['dist_b1'][None])
        ga, gc = _sc_gather(ua, vc, ei2, ej2)
        oute = _edge_call(
            ga, gc, p['dist_w2'].T, p['ctx_w1'][:, :D].T, p['ctx_w2'].T,
            p['dist_gnw'][None], p['dist_gnb'][None],
            p['ctx_gnw'][None], p['ctx_gnb'][None])
        acc = _sc_scatter(oute, sidx2, zrows)
        agts = _post_call(
            agts, acc[0], acc[1], p['agt_w'].T, p['norm_w'][None],
            p['norm_b'][None], p['lin_w'].T, p['lin_gnw'][None],
            p['lin_gnb'][None])
    return agts[:NA]
